# baseline (speedup 1.0000x reference)
import functools

import numpy as np
import jax
import jax.numpy as jnp
from jax import lax
from jax.experimental import pallas as pl
from jax.experimental.pallas import tpu as pltpu
from jax.experimental.pallas import tpu_sc as plsc

F = 26
V = 100000
D = 32
B = 4096
H1 = 512
H2 = 256
NC, NS = 2, 16
FPC = F // NC
BPT = B // NS
NPT = FPC * BPT
SH_WORDS = 1302528
ZCH = SH_WORDS // NS
ZBUF = 10176

_EPS = 1e-5
_TRIU_R, _TRIU_C = np.triu_indices(F, k=1)


def _sc_hist(xt):
    @functools.partial(
        pl.kernel,
        mesh=plsc.VectorSubcoreMesh(core_axis_name="c", subcore_axis_name="s"),
        out_type=jax.ShapeDtypeStruct((NC * SH_WORDS,), jnp.float32),
        compiler_params=pltpu.CompilerParams(needs_layout_passes=False),
        scratch_types=[
            pltpu.VMEM((ZBUF,), jnp.float32),
            pltpu.VMEM((FPC, BPT // 128, 128), jnp.int32),
            pltpu.VMEM((128,), jnp.int32),
            pltpu.VMEM((ZBUF,), jnp.float32),
            pltpu.VMEM((128,), jnp.float32),
            pltpu.VMEM_SHARED((SH_WORDS,), jnp.float32),
            pltpu.SemaphoreType.DMA,
        ],
    )
    def k(xt_hbm, out_hbm, zbuf, idxs, idx1, sbuf, ones_v, hshared, sem):
        sc = lax.axis_index("c")
        tl = lax.axis_index("s")

        for g in range(ZBUF // 16):
            zbuf[pl.ds(g * 16, 16)] = jnp.zeros((16,), jnp.float32)
        for g in range(8):
            ones_v[pl.ds(g * 16, 16)] = jnp.full((16,), 1.0, jnp.float32)

        pltpu.sync_copy(
            xt_hbm.at[pl.ds(sc * FPC, FPC), pl.ds(tl * (BPT // 128),
                                                  BPT // 128)], idxs)
        zbase = tl * ZCH
        for i in range(ZCH // ZBUF):
            pltpu.sync_copy(zbuf, hshared.at[pl.ds(zbase + i * ZBUF, ZBUF)])
        plsc.subcore_barrier()

        for fl in range(FPC):
            off = fl * V
            for r in range(BPT // 128):
                for g in range(8):
                    sl = pl.ds(g * 16, 16)
                    idx1[sl] = idxs[fl, r, sl] + off
                pltpu.sync_copy(ones_v, hshared.at[idx1], add=True)
        plsc.subcore_barrier()

        obase = sc * SH_WORDS + tl * ZCH
        for i in range(ZCH // ZBUF):
            pltpu.sync_copy(hshared.at[pl.ds(tl * ZCH + i * ZBUF, ZBUF)], sbuf)
            pltpu.sync_copy(sbuf, out_hbm.at[pl.ds(obase + i * ZBUF, ZBUF)])

    return k(xt)


_VB = 5000
_NJ = (V // 4) // _VB


def _hmm_body(h_ref, t_ref, o_ref, acc_ref):
    j = pl.program_id(1)

    @pl.when(j == 0)
    def _():
        acc_ref[...] = jnp.zeros_like(acc_ref)

    acc_ref[...] += lax.dot_general(
        h_ref[0], t_ref[0], (((0,), (0,)), ((), ())),
        preferred_element_type=jnp.float32)

    @pl.when(j == _NJ - 1)
    def _():
        acc = acc_ref[...]
        o_ref[0] = (acc[0:1, 0:D] + acc[1:2, D:2 * D]
                    + acc[2:3, 2 * D:3 * D] + acc[3:4, 3 * D:4 * D])


def _tc_hist_matmul(hist4, tab128):
    return pl.pallas_call(
        _hmm_body,
        grid=(F, _NJ),
        in_specs=[
            pl.BlockSpec((1, _VB, 4), lambda k, j: (k, j, 0)),
            pl.BlockSpec((1, _VB, 128), lambda k, j: (k, j, 0)),
        ],
        out_specs=pl.BlockSpec((1, 1, D), lambda k, j: (k, 0, 0)),
        out_shape=jax.ShapeDtypeStruct((F, 1, D), jnp.float32),
        scratch_shapes=[pltpu.VMEM((4, 128), jnp.float32)],
    )(hist4, tab128)


def _dense_body(s_ref, w1_ref, wg_ref, g1_ref, be1_ref, w2_ref,
                g2_ref, be2_ref, wo_ref, bo_ref, o_ref):
    S = s_ref[:]
    G = lax.dot_general(S, S, (((1,), (1,)), ((), ())),
                        preferred_element_type=jnp.float32,
                        precision=lax.Precision.HIGHEST)
    Gf = jnp.concatenate([G[i:i + 1, :] for i in range(F)], axis=1)
    Sf = jnp.concatenate([S[i:i + 1, :] for i in range(F)], axis=1)
    d = (jnp.dot(Sf, w1_ref[:], preferred_element_type=jnp.float32,
                 precision=lax.Precision.HIGHEST)
         + jnp.dot(Gf, wg_ref[:], preferred_element_type=jnp.float32,
                   precision=lax.Precision.HIGHEST))
    fB = float(B)
    alpha = (fB - 1.0) / (fB * fB)
    rs = lax.rsqrt(d * d * alpha + _EPS)
    g1 = g1_ref[:]
    be1 = be1_ref[:]
    u = jnp.maximum(be1 - (d * (1.0 / fB)) * rs * g1, 0.0)
    w = jnp.maximum(be1 + (d * ((fB - 1.0) / fB)) * rs * g1, 0.0)
    e = jnp.dot(w - u, w2_ref[:], preferred_element_type=jnp.float32,
                precision=lax.Precision.HIGHEST)
    rs2 = lax.rsqrt(e * e * alpha + _EPS)
    g2 = g2_ref[:]
    be2 = be2_ref[:]
    u2 = jnp.maximum(be2 - (e * (1.0 / fB)) * rs2 * g2, 0.0)
    w2 = jnp.maximum(be2 + (e * ((fB - 1.0) / fB)) * rs2 * g2, 0.0)
    wo = wo_ref[:]
    bo = bo_ref[0, 0]
    oc = jnp.sum(u2 * wo) + bo
    os_ = jnp.sum(w2 * wo) + bo
    lin = (lax.broadcasted_iota(jnp.int32, (B // 128, 128), 0) * 128
           + lax.broadcasted_iota(jnp.int32, (B // 128, 128), 1))
    logits = jnp.where(lin == B - 1, jnp.full((B // 128, 128), os_),
                       jnp.full((B // 128, 128), oc))
    o_ref[:] = jax.nn.sigmoid(logits)


def _tc_dense(S, W1eT, WgT, g1, be1, W2T, g2, be2, Wout, bout):
    return pl.pallas_call(
        _dense_body,
        out_shape=jax.ShapeDtypeStruct((B // 128, 128), jnp.float32),
    )(S, W1eT, WgT, g1, be1, W2T, g2, be2, Wout, bout)


def kernel(x, tables, W1, b1, g1, be1, W2, b2, g2, be2, Wout, bout):
    xt = x.T.reshape(F, B // 128, 128)
    hist = _sc_hist(xt).reshape(NC, SH_WORDS)[:, :FPC * V]
    hist4 = hist.reshape(F, V // 4, 4)
    tab128 = tables.reshape(F, V // 4, 128)
    S = _tc_hist_matmul(hist4, tab128).reshape(F, D)

    W1eT = W1[:, :F * D].T
    WgT = jnp.zeros((F * F, H1), jnp.float32).at[
        _TRIU_R * F + _TRIU_C, :].set(W1[:, F * D:].T)
    out2d = _tc_dense(
        S, W1eT, WgT,
        g1.reshape(1, H1), be1.reshape(1, H1),
        W2.T, g2.reshape(1, H2), be2.reshape(1, H2),
        Wout, bout.reshape(1, 1),
    )
    return out2d.reshape(B)


# speedup vs baseline: 1.4591x; 1.4591x over previous
import functools

import numpy as np
import jax
import jax.numpy as jnp
from jax import lax
from jax.experimental import pallas as pl
from jax.experimental.pallas import tpu as pltpu
from jax.experimental.pallas import tpu_sc as plsc

F = 26
V = 100000
D = 32
B = 4096
H1 = 512
H2 = 256
C = 8
RSUB = 4
NT = F * C
NC, NS = 2, 16
NW = NC * NS

_EPS = 1e-5
_TRIU_R, _TRIU_C = np.triu_indices(F, k=1)


def _sc_partial_sums(xt, tab128):
    @functools.partial(
        pl.kernel,
        mesh=plsc.VectorSubcoreMesh(core_axis_name="c", subcore_axis_name="s"),
        out_type=jax.ShapeDtypeStruct((NT, D), jnp.float32),
        compiler_params=pltpu.CompilerParams(needs_layout_passes=False),
        scratch_types=[
            pltpu.VMEM((RSUB * 128,), jnp.int32),
            pltpu.VMEM((RSUB * 128,), jnp.int32),
            pltpu.VMEM((RSUB * 128, 128), jnp.float32),
            pltpu.VMEM((D,), jnp.float32),
            pltpu.SemaphoreType.DMA,
        ],
    )
    def k(xt_hbm, tab_hbm, out_hbm, idx_v, offs_v, rows_v, obuf_v, sem):
        wid = lax.axis_index("s") * NC + lax.axis_index("c")
        iota16 = lax.iota(jnp.int32, 16)

        def do_task(t):
            fk = t // C
            ck = t % C
            pltpu.sync_copy(xt_hbm.at[fk, ck], idx_v)
            roff = fk * (V // 4)

            for q in range(RSUB * 8):
                sl = pl.ds(q * 16, 16)
                v = idx_v[sl]
                offs_v[sl] = lax.shift_left(v & 3, 5)
                idx_v[sl] = lax.shift_right_logical(v, 2) + roff
            cps = [
                pltpu.async_copy(
                    tab_hbm.at[idx_v.at[pl.ds(j * 128, 128)]],
                    rows_v.at[pl.ds(j * 128, 128), :], sem)
                for j in range(RSUB)
            ]
            for cp in cps:
                cp.wait()
            a0 = jnp.zeros((16,), jnp.float32)
            a1 = jnp.zeros((16,), jnp.float32)

            def body(r4, accs):
                b0, b1 = accs
                for l in range(4):
                    r = r4 * 4 + l
                    rsplat = jnp.full((16,), r, jnp.int32)
                    mo = plsc.load_gather(offs_v, [rsplat])
                    c0 = mo + iota16
                    b0 = b0 + plsc.load_gather(rows_v, [rsplat, c0])
                    b1 = b1 + plsc.load_gather(rows_v, [rsplat, c0 + 16])
                return (b0, b1)
            a0, a1 = lax.fori_loop(0, RSUB * 32, body, (a0, a1))
            obuf_v[pl.ds(0, 16)] = a0
            obuf_v[pl.ds(16, 16)] = a1
            pltpu.sync_copy(obuf_v, out_hbm.at[ck * F + fk])

        for i in range((NT + NW - 1) // NW):
            t = wid + NW * i
            if (i + 1) * NW <= NT:
                do_task(t)
            else:
                @pl.when(t < NT)
                def _():
                    do_task(t)

    return k(xt, tab128)


def _dense_body(p_ref, p4_ref, w1_ref, wg_ref, g1_ref, be1_ref, w2_ref,
                g2_ref, be2_ref, wo_ref, bo_ref, o_ref):
    P = p_ref[:]
    S = P[0:F]
    for c in range(1, C):
        S = S + P[c * F:(c + 1) * F]
    G = lax.dot_general(S, S, (((1,), (1,)), ((), ())),
                        preferred_element_type=jnp.float32, precision=lax.Precision.HIGHEST)
    Gf = jnp.concatenate([G[i:i + 1, :] for i in range(F)], axis=1)
    d4 = jnp.dot(p4_ref[:], w1_ref[:], preferred_element_type=jnp.float32, precision=lax.Precision.HIGHEST)
    d = (jnp.sum(d4, axis=0, keepdims=True)
         + jnp.dot(Gf, wg_ref[:], preferred_element_type=jnp.float32, precision=lax.Precision.HIGHEST))
    fB = float(B)
    alpha = (fB - 1.0) / (fB * fB)
    rs = lax.rsqrt(d * d * alpha + _EPS)
    g1 = g1_ref[:]
    be1 = be1_ref[:]
    u = jnp.maximum(be1 - (d * (1.0 / fB)) * rs * g1, 0.0)
    w = jnp.maximum(be1 + (d * ((fB - 1.0) / fB)) * rs * g1, 0.0)
    e = jnp.dot(w - u, w2_ref[:], preferred_element_type=jnp.float32, precision=lax.Precision.HIGHEST)
    rs2 = lax.rsqrt(e * e * alpha + _EPS)
    g2 = g2_ref[:]
    be2 = be2_ref[:]
    u2 = jnp.maximum(be2 - (e * (1.0 / fB)) * rs2 * g2, 0.0)
    w2 = jnp.maximum(be2 + (e * ((fB - 1.0) / fB)) * rs2 * g2, 0.0)
    wo = wo_ref[:]
    bo = bo_ref[0, 0]
    oc = jnp.sum(u2 * wo) + bo
    os_ = jnp.sum(w2 * wo) + bo
    lin = (lax.broadcasted_iota(jnp.int32, (B // 128, 128), 0) * 128
           + lax.broadcasted_iota(jnp.int32, (B // 128, 128), 1))
    logits = jnp.where(lin == B - 1, jnp.full((B // 128, 128), os_),
                       jnp.full((B // 128, 128), oc))
    o_ref[:] = jax.nn.sigmoid(logits)


def _tc_dense(partials, part4, W1eT, WgT, g1, be1, W2T, g2, be2, Wout, bout):
    return pl.pallas_call(
        _dense_body,
        out_shape=jax.ShapeDtypeStruct((B // 128, 128), jnp.float32),
    )(partials, part4, W1eT, WgT, g1, be1, W2T, g2, be2, Wout, bout)


def kernel(x, tables, W1, b1, g1, be1, W2, b2, g2, be2, Wout, bout):
    xt = x.T.reshape(F, C, RSUB * 128)
    tab128 = tables.reshape(F * V // 4, 128)
    partials = _sc_partial_sums(xt, tab128)
    part4 = partials.reshape(C, F * D)

    W1eT = W1[:, :F * D].T
    WgT = jnp.zeros((F * F, H1), jnp.float32).at[
        _TRIU_R * F + _TRIU_C, :].set(W1[:, F * D:].T)
    out2d = _tc_dense(
        partials, part4, W1eT, WgT,
        g1.reshape(1, H1), be1.reshape(1, H1),
        W2.T, g2.reshape(1, H2), be2.reshape(1, H2),
        Wout, bout.reshape(1, 1),
    )
    return out2d.reshape(B)
